# baseline (speedup 1.0000x reference)
import jax
import jax.numpy as jnp
from jax.experimental import pallas as pl
from jax.experimental.pallas import tpu as pltpu

_TOKENS = 32768
_D = 4096
_E = 64
_K = 8
_BT = 1024
_NEG = float("-inf")


def _router_block(x_ref, w_ref, b_ref, idx_ref, probs_ref):
    xb = x_ref[...]
    logits = jax.lax.dot_general(
        w_ref[...], xb, (((1,), (1,)), ((), ())),
        preferred_element_type=jnp.float32,
    ) + b_ref[...]
    iota = jax.lax.broadcasted_iota(jnp.int32, logits.shape, 0)
    work = logits
    idx_rows = []
    m0 = None
    for k in range(_K):
        m = jnp.max(work, axis=0, keepdims=True)
        if k == 0:
            m0 = m
        idxk = jnp.min(jnp.where(work == m, iota, _E), axis=0, keepdims=True)
        chosen = iota == idxk
        work = jnp.where(chosen, _NEG, work)
        idx_rows.append(idxk)
    idx_ref[...] = jnp.concatenate(idx_rows, axis=0)
    sel = work == _NEG
    e = jnp.where(sel, jnp.exp(logits - m0), jnp.float32(0.0))
    probs_ref[...] = e / jnp.sum(e, axis=0, keepdims=True)


def kernel(x, W, b):
    b2 = b.reshape(_E, 1)
    grid = (_TOKENS // _BT,)
    idx_t, probs_t = pl.pallas_call(
        _router_block,
        grid=grid,
        in_specs=[
            pl.BlockSpec((_BT, _D), lambda i: (i, 0)),
            pl.BlockSpec((_E, _D), lambda i: (0, 0)),
            pl.BlockSpec((_E, 1), lambda i: (0, 0)),
        ],
        out_specs=[
            pl.BlockSpec((_K, _BT), lambda i: (0, i)),
            pl.BlockSpec((_E, _BT), lambda i: (0, i)),
        ],
        out_shape=[
            jax.ShapeDtypeStruct((_K, _TOKENS), jnp.int32),
            jax.ShapeDtypeStruct((_E, _TOKENS), jnp.float32),
        ],
        compiler_params=pltpu.CompilerParams(
            dimension_semantics=("parallel",),
        ),
    )(x, W, b2)
    return idx_t.T, probs_t.T


# speedup vs baseline: 1.0001x; 1.0001x over previous
import jax
import jax.numpy as jnp
from jax.experimental import pallas as pl
from jax.experimental.pallas import tpu as pltpu

_TOKENS = 32768
_D = 4096
_E = 64
_K = 8
_BT = 1024
_NEG = float("-inf")


def _router_block(x1_ref, x2_ref, w1_ref, w2_ref, b_ref, idx_ref, probs_ref):
    logits = (
        jax.lax.dot_general(
            w1_ref[...], x1_ref[...], (((1,), (1,)), ((), ())),
            preferred_element_type=jnp.float32,
        )
        + jax.lax.dot_general(
            w2_ref[...], x2_ref[...], (((1,), (1,)), ((), ())),
            preferred_element_type=jnp.float32,
        )
        + b_ref[...]
    )
    iota = jax.lax.broadcasted_iota(jnp.int32, logits.shape, 0)
    work = logits
    idx_rows = []
    m0 = None
    for k in range(_K):
        m = jnp.max(work, axis=0, keepdims=True)
        if k == 0:
            m0 = m
        idxk = jnp.min(jnp.where(work == m, iota, _E), axis=0, keepdims=True)
        chosen = iota == idxk
        work = jnp.where(chosen, _NEG, work)
        idx_rows.append(idxk)
    idx_ref[...] = jnp.concatenate(idx_rows, axis=0)
    sel = work == _NEG
    e = jnp.where(sel, jnp.exp(logits - m0), jnp.float32(0.0))
    probs_ref[...] = e / jnp.sum(e, axis=0, keepdims=True)


def kernel(x, W, b):
    b2 = b.reshape(_E, 1)
    grid = (_TOKENS // _BT,)
    idx_t, probs_t = pl.pallas_call(
        _router_block,
        grid=grid,
        in_specs=[
            pl.BlockSpec((_BT, _D // 2), lambda i: (i, 0)),
            pl.BlockSpec((_BT, _D // 2), lambda i: (i, 1)),
            pl.BlockSpec((_E, _D // 2), lambda i: (0, 0)),
            pl.BlockSpec((_E, _D // 2), lambda i: (0, 1)),
            pl.BlockSpec((_E, 1), lambda i: (0, 0)),
        ],
        out_specs=[
            pl.BlockSpec((_K, _BT), lambda i: (0, i)),
            pl.BlockSpec((_E, _BT), lambda i: (0, i)),
        ],
        out_shape=[
            jax.ShapeDtypeStruct((_K, _TOKENS), jnp.int32),
            jax.ShapeDtypeStruct((_E, _TOKENS), jnp.float32),
        ],
        compiler_params=pltpu.CompilerParams(
            dimension_semantics=("parallel",),
        ),
    )(x, x, W, W, b2)
    return idx_t.T, probs_t.T


# speedup vs baseline: 1.0004x; 1.0003x over previous
import jax
import jax.numpy as jnp
from jax.experimental import pallas as pl
from jax.experimental.pallas import tpu as pltpu

_TOKENS = 32768
_D = 4096
_E = 64
_K = 8
_BT = 1024
_NEG = float("-inf")


def _router_block(x_ref, w_ref, b_ref, idx_ref, probs_ref):
    xb = x_ref[...]
    logits = jax.lax.dot_general(
        w_ref[...], xb, (((1,), (1,)), ((), ())),
        preferred_element_type=jnp.float32,
    ) + b_ref[...]
    iota = jax.lax.broadcasted_iota(jnp.int32, logits.shape, 0)
    work = logits
    idx_rows = []
    m0 = None
    for k in range(_K):
        m = jnp.max(work, axis=0, keepdims=True)
        if k == 0:
            m0 = m
        idxk = jnp.min(jnp.where(work == m, iota, _E), axis=0, keepdims=True)
        chosen = iota == idxk
        work = jnp.where(chosen, _NEG, work)
        idx_rows.append(idxk)
    idx_ref[...] = jnp.concatenate(idx_rows, axis=0)
    sel = work == _NEG
    e = jnp.where(sel, jnp.exp(logits - m0), jnp.float32(0.0))
    probs_ref[...] = e / jnp.sum(e, axis=0, keepdims=True)


def kernel(x, W, b):
    b2 = b.reshape(_E, 1)
    grid = (_TOKENS // _BT,)
    idx_t, probs_t = pl.pallas_call(
        _router_block,
        grid=grid,
        in_specs=[
            pl.BlockSpec((_BT, _D), lambda i: (i, 0)),
            pl.BlockSpec((_E, _D), lambda i: (0, 0)),
            pl.BlockSpec((_E, 1), lambda i: (0, 0)),
        ],
        out_specs=[
            pl.BlockSpec((_K, _BT), lambda i: (0, i)),
            pl.BlockSpec((_E, _BT), lambda i: (0, i)),
        ],
        out_shape=[
            jax.ShapeDtypeStruct((_K, _TOKENS), jnp.int32),
            jax.ShapeDtypeStruct((_E, _TOKENS), jnp.float32),
        ],
        compiler_params=pltpu.CompilerParams(
            dimension_semantics=("parallel",),
        ),
    )(x, W, b2)
    return idx_t.T, probs_t.T
